# initial kernel scaffold (unmeasured)
import jax
import jax.numpy as jnp
from jax import lax
from jax.experimental import pallas as pl
from jax.experimental.pallas import tpu as pltpu

N_DEV = 32


def _ring_all_reduce(partial):
    m, n = partial.shape
    chunk = m // N_DEV

    def body(p_ref, out_ref, acc_ref, recv_ref, local_ref,
             send_sem, recv_sem, load_sem, store_sem, credit_sem):
        my = lax.axis_index("i")
        left = lax.rem(my + N_DEV - 1, N_DEV)
        right = lax.rem(my + 1, N_DEV)

        barrier = pltpu.get_barrier_semaphore()
        pl.semaphore_signal(barrier, inc=1, device_id=(left,),
                            device_id_type=pl.DeviceIdType.MESH)
        pl.semaphore_signal(barrier, inc=1, device_id=(right,),
                            device_id_type=pl.DeviceIdType.MESH)
        pl.semaphore_wait(barrier, 2)

        ld0 = pltpu.make_async_copy(
            p_ref.at[pl.ds(my * chunk, chunk), :], acc_ref, load_sem)
        ld0.start()
        ld0.wait()

        def make_rdma():
            return pltpu.make_async_remote_copy(
                src_ref=acc_ref, dst_ref=recv_ref,
                send_sem=send_sem, recv_sem=recv_sem,
                device_id=(right,), device_id_type=pl.DeviceIdType.MESH)

        def rs_step(s, carry):
            idx = lax.rem(my - s - 1 + 2 * N_DEV, N_DEV)
            ld = pltpu.make_async_copy(
                p_ref.at[pl.ds(idx * chunk, chunk), :], local_ref, load_sem)
            ld.start()

            @pl.when(s > 0)
            def _():
                pl.semaphore_wait(credit_sem, 1)

            rdma = make_rdma()
            rdma.start()
            rdma.wait()
            ld.wait()
            acc_ref[...] = recv_ref[...] + local_ref[...]
            pl.semaphore_signal(credit_sem, inc=1, device_id=(left,),
                                device_id_type=pl.DeviceIdType.MESH)
            return carry

        lax.fori_loop(0, N_DEV - 1, rs_step, 0)

        owned = lax.rem(my + 1, N_DEV)
        st0 = pltpu.make_async_copy(
            acc_ref, out_ref.at[pl.ds(owned * chunk, chunk), :], store_sem)
        st0.start()
        st0.wait()

        def ag_step(s, carry):
            pl.semaphore_wait(credit_sem, 1)
            rdma = make_rdma()
            rdma.start()
            rdma.wait()
            idx = lax.rem(my - s + 2 * N_DEV, N_DEV)
            st = pltpu.make_async_copy(
                recv_ref, out_ref.at[pl.ds(idx * chunk, chunk), :], store_sem)
            st.start()
            acc_ref[...] = recv_ref[...]
            st.wait()

            @pl.when(s < N_DEV - 2)
            def _():
                pl.semaphore_signal(credit_sem, inc=1, device_id=(left,),
                                    device_id_type=pl.DeviceIdType.MESH)
            return carry

        lax.fori_loop(0, N_DEV - 1, ag_step, 0)

    return pl.pallas_call(
        body,
        out_shape=jax.ShapeDtypeStruct((m, n), jnp.float32),
        in_specs=[pl.BlockSpec(memory_space=pltpu.ANY)],
        out_specs=pl.BlockSpec(memory_space=pltpu.ANY),
        scratch_shapes=[
            pltpu.VMEM((chunk, n), jnp.float32),
            pltpu.VMEM((chunk, n), jnp.float32),
            pltpu.VMEM((chunk, n), jnp.float32),
            pltpu.SemaphoreType.DMA,
            pltpu.SemaphoreType.DMA,
            pltpu.SemaphoreType.DMA,
            pltpu.SemaphoreType.DMA,
            pltpu.SemaphoreType.REGULAR,
        ],
        compiler_params=pltpu.CompilerParams(collective_id=0),
    )(partial)


def kernel(x, w_mat):
    partial = jnp.dot(x, w_mat, preferred_element_type=jnp.float32,
                      precision=lax.Precision.HIGHEST)
    y = _ring_all_reduce(partial)
    amax = jnp.max(jnp.abs(y))
    scale = amax / 127.0
    q = jnp.clip(jnp.round(y / scale), -127.0, 127.0)
    return (q * scale).astype(jnp.float32)


# baseline (device time: 3548737 ns/iter reference)
import jax
import jax.numpy as jnp
from jax import lax
from jax.experimental import pallas as pl
from jax.experimental.pallas import tpu as pltpu

N_DEV = 32


def _ring_all_reduce(partial):
    m, n = partial.shape
    chunk = m // N_DEV

    def body(p_ref, out_ref, acc_ref, recv_ref, local_ref,
             send_sem, recv_sem, load_sem, store_sem, credit_sem):
        my = lax.axis_index("i")
        left = lax.rem(my + N_DEV - 1, N_DEV)
        right = lax.rem(my + 1, N_DEV)

        barrier = pltpu.get_barrier_semaphore()
        pl.semaphore_signal(barrier, inc=1, device_id=(left,),
                            device_id_type=pl.DeviceIdType.MESH)
        pl.semaphore_signal(barrier, inc=1, device_id=(right,),
                            device_id_type=pl.DeviceIdType.MESH)
        pl.semaphore_wait(barrier, 2)

        ld0 = pltpu.make_async_copy(
            p_ref.at[pl.ds(my * chunk, chunk), :], acc_ref, load_sem)
        ld0.start()
        ld0.wait()

        def make_rdma():
            return pltpu.make_async_remote_copy(
                src_ref=acc_ref, dst_ref=recv_ref,
                send_sem=send_sem, recv_sem=recv_sem,
                device_id=(right,), device_id_type=pl.DeviceIdType.MESH)

        def rs_step(s, carry):
            idx = lax.rem(my - s - 1 + 2 * N_DEV, N_DEV)
            ld = pltpu.make_async_copy(
                p_ref.at[pl.ds(idx * chunk, chunk), :], local_ref, load_sem)
            ld.start()

            @pl.when(s > 0)
            def _():
                pl.semaphore_wait(credit_sem, 1)

            rdma = make_rdma()
            rdma.start()
            rdma.wait()
            ld.wait()
            acc_ref[...] = recv_ref[...] + local_ref[...]
            pl.semaphore_signal(credit_sem, inc=1, device_id=(left,),
                                device_id_type=pl.DeviceIdType.MESH)
            return carry

        lax.fori_loop(0, N_DEV - 1, rs_step, 0)

        owned = lax.rem(my + 1, N_DEV)
        st0 = pltpu.make_async_copy(
            acc_ref, out_ref.at[pl.ds(owned * chunk, chunk), :], store_sem)
        st0.start()
        st0.wait()

        def ag_step(s, carry):
            pl.semaphore_wait(credit_sem, 1)
            rdma = make_rdma()
            rdma.start()
            rdma.wait()
            idx = lax.rem(my - s + 2 * N_DEV, N_DEV)
            st = pltpu.make_async_copy(
                recv_ref, out_ref.at[pl.ds(idx * chunk, chunk), :], store_sem)
            st.start()
            acc_ref[...] = recv_ref[...]
            st.wait()

            @pl.when(s < N_DEV - 2)
            def _():
                pl.semaphore_signal(credit_sem, inc=1, device_id=(left,),
                                    device_id_type=pl.DeviceIdType.MESH)
            return carry

        lax.fori_loop(0, N_DEV - 1, ag_step, 0)

    return pl.pallas_call(
        body,
        out_shape=jax.ShapeDtypeStruct((m, n), jnp.float32),
        in_specs=[pl.BlockSpec(memory_space=pl.ANY)],
        out_specs=pl.BlockSpec(memory_space=pl.ANY),
        scratch_shapes=[
            pltpu.VMEM((chunk, n), jnp.float32),
            pltpu.VMEM((chunk, n), jnp.float32),
            pltpu.VMEM((chunk, n), jnp.float32),
            pltpu.SemaphoreType.DMA,
            pltpu.SemaphoreType.DMA,
            pltpu.SemaphoreType.DMA,
            pltpu.SemaphoreType.DMA,
            pltpu.SemaphoreType.REGULAR,
        ],
        compiler_params=pltpu.CompilerParams(collective_id=0),
    )(partial)


def kernel(x, w_mat):
    partial = jnp.dot(x, w_mat, preferred_element_type=jnp.float32,
                      precision=lax.Precision.HIGHEST)
    y = _ring_all_reduce(partial)
    amax = jnp.max(jnp.abs(y))
    scale = amax / 127.0
    q = jnp.clip(jnp.round(y / scale), -127.0, 127.0)
    return (q * scale).astype(jnp.float32)


# device time: 3259621 ns/iter; 1.0887x vs baseline; 1.0887x over previous
import jax
import jax.numpy as jnp
from jax import lax
from jax.experimental import pallas as pl
from jax.experimental.pallas import tpu as pltpu

N_DEV = 32
SGN = (1, -1)


def _ring_all_reduce(partial):
    m, n = partial.shape
    chunk = m // N_DEV
    half = n // 2

    def body(p_ref, out_ref,
             acc0, recv0, local0, acc1, recv1, local1,
             send_sem0, send_sem1, recv_sem0, recv_sem1,
             load_sem0, load_sem1, store_sem0, store_sem1,
             credit0, credit1):
        acc = (acc0, acc1)
        recv = (recv0, recv1)
        local = (local0, local1)
        send_sem = (send_sem0, send_sem1)
        recv_sem = (recv_sem0, recv_sem1)
        load_sem = (load_sem0, load_sem1)
        store_sem = (store_sem0, store_sem1)
        credit = (credit0, credit1)

        my = lax.axis_index("i")
        left = lax.rem(my + N_DEV - 1, N_DEV)
        right = lax.rem(my + 1, N_DEV)
        nbr_send = (right, left)
        nbr_recv = (left, right)

        barrier = pltpu.get_barrier_semaphore()
        for nbr in (left, right):
            pl.semaphore_signal(barrier, inc=1, device_id=(nbr,),
                                device_id_type=pl.DeviceIdType.MESH)
        pl.semaphore_wait(barrier, 2)

        def cols(d):
            return pl.ds(d * half, half)

        seeds = []
        for d in (0, 1):
            ld = pltpu.make_async_copy(
                p_ref.at[pl.ds(my * chunk, chunk), cols(d)],
                acc[d], load_sem[d])
            ld.start()
            seeds.append(ld)
        for ld in seeds:
            ld.wait()

        def make_rdma(d):
            return pltpu.make_async_remote_copy(
                src_ref=acc[d], dst_ref=recv[d],
                send_sem=send_sem[d], recv_sem=recv_sem[d],
                device_id=(nbr_send[d],),
                device_id_type=pl.DeviceIdType.MESH)

        def rs_step(s, carry):
            lds = []
            for d in (0, 1):
                idx = lax.rem(my - SGN[d] * (s + 1) + 2 * N_DEV, N_DEV)
                ld = pltpu.make_async_copy(
                    p_ref.at[pl.ds(idx * chunk, chunk), cols(d)],
                    local[d], load_sem[d])
                ld.start()
                lds.append(ld)

            @pl.when(s > 0)
            def _():
                for d in (0, 1):
                    pl.semaphore_wait(credit[d], 1)

            rdmas = [make_rdma(d) for d in (0, 1)]
            for r in rdmas:
                r.start()
            for r in rdmas:
                r.wait()
            for d in (0, 1):
                lds[d].wait()
                acc[d][...] = recv[d][...] + local[d][...]
                pl.semaphore_signal(credit[d], inc=1,
                                    device_id=(nbr_recv[d],),
                                    device_id_type=pl.DeviceIdType.MESH)
            return carry

        lax.fori_loop(0, N_DEV - 1, rs_step, 0)

        sts = []
        for d in (0, 1):
            owned = lax.rem(my + SGN[d] + N_DEV, N_DEV)
            st = pltpu.make_async_copy(
                acc[d], out_ref.at[pl.ds(owned * chunk, chunk), cols(d)],
                store_sem[d])
            st.start()
            sts.append(st)
        for st in sts:
            st.wait()

        def ag_step(s, carry):
            for d in (0, 1):
                pl.semaphore_wait(credit[d], 1)
            rdmas = [make_rdma(d) for d in (0, 1)]
            for r in rdmas:
                r.start()
            for r in rdmas:
                r.wait()
            sts = []
            for d in (0, 1):
                idx = lax.rem(my - SGN[d] * s + 2 * N_DEV, N_DEV)
                st = pltpu.make_async_copy(
                    recv[d], out_ref.at[pl.ds(idx * chunk, chunk), cols(d)],
                    store_sem[d])
                st.start()
                acc[d][...] = recv[d][...]
                sts.append(st)
            for st in sts:
                st.wait()

            @pl.when(s < N_DEV - 2)
            def _():
                for d in (0, 1):
                    pl.semaphore_signal(credit[d], inc=1,
                                        device_id=(nbr_recv[d],),
                                        device_id_type=pl.DeviceIdType.MESH)
            return carry

        lax.fori_loop(0, N_DEV - 1, ag_step, 0)

    buf = lambda: pltpu.VMEM((chunk, half), jnp.float32)
    return pl.pallas_call(
        body,
        out_shape=jax.ShapeDtypeStruct((m, n), jnp.float32),
        in_specs=[pl.BlockSpec(memory_space=pl.ANY)],
        out_specs=pl.BlockSpec(memory_space=pl.ANY),
        scratch_shapes=[
            buf(), buf(), buf(),
            buf(), buf(), buf(),
            pltpu.SemaphoreType.DMA, pltpu.SemaphoreType.DMA,
            pltpu.SemaphoreType.DMA, pltpu.SemaphoreType.DMA,
            pltpu.SemaphoreType.DMA, pltpu.SemaphoreType.DMA,
            pltpu.SemaphoreType.DMA, pltpu.SemaphoreType.DMA,
            pltpu.SemaphoreType.REGULAR, pltpu.SemaphoreType.REGULAR,
        ],
        compiler_params=pltpu.CompilerParams(collective_id=0),
    )(partial)


def kernel(x, w_mat):
    partial = jnp.dot(x, w_mat, preferred_element_type=jnp.float32,
                      precision=lax.Precision.HIGHEST)
    y = _ring_all_reduce(partial)
    amax = jnp.max(jnp.abs(y))
    scale = amax / 127.0
    q = jnp.clip(jnp.round(y / scale), -127.0, 127.0)
    return (q * scale).astype(jnp.float32)


# device time: 1877159 ns/iter; 1.8905x vs baseline; 1.7365x over previous
import numpy as np

import jax
import jax.numpy as jnp
from jax import lax
from jax.experimental import pallas as pl
from jax.experimental.pallas import tpu as pltpu

N_DEV = 32
SGN = (1, -1)


def _mesh_coord_order():
    coords = [(x, y, z) for x in (0, 1) for y in range(4) for z in range(4)]
    order = []
    for z in sorted({c[2] for c in coords}):
        plane = sorted(c for c in coords if c[2] == z)
        for yi, y in enumerate(sorted({c[1] for c in plane})):
            row = sorted((c for c in plane if c[1] == y), reverse=bool(yi % 2))
            order.extend(row)
    return order


def _ham_cycle():
    path = []
    for zi in range(4):
        ys = range(4) if zi % 2 == 0 else range(3, -1, -1)
        for y in ys:
            path.append((0, y, zi))
    path += [(1, y, z) for (_, y, z) in reversed(path)]
    return path


_MESH_ORDER = _mesh_coord_order()
_HAM = _ham_cycle()
for _k in range(N_DEV):
    _a, _b = _HAM[_k], _HAM[(_k + 1) % N_DEV]
    assert sum(abs(_a[_i] - _b[_i]) for _i in range(3)) == 1, (_k, _a, _b)
PERM = np.array([_MESH_ORDER.index(c) for c in _HAM], dtype=np.int32)
assert sorted(PERM.tolist()) == list(range(N_DEV))
INV = np.argsort(PERM).astype(np.int32)


def _ring_all_reduce(partial, meta):
    m, n = partial.shape
    chunk = m // N_DEV
    half = n // 2

    def body(meta_ref, p_ref, out_ref,
             acc0, recv0, local0, acc1, recv1, local1,
             send_sem0, send_sem1, recv_sem0, recv_sem1,
             load_sem0, load_sem1, store_sem0, store_sem1,
             credit0, credit1):
        acc = (acc0, acc1)
        recv = (recv0, recv1)
        local = (local0, local1)
        send_sem = (send_sem0, send_sem1)
        recv_sem = (recv_sem0, recv_sem1)
        load_sem = (load_sem0, load_sem1)
        store_sem = (store_sem0, store_sem1)
        credit = (credit0, credit1)

        rr = meta_ref[0]
        left = meta_ref[1]
        right = meta_ref[2]
        nbr_send = (right, left)
        nbr_recv = (left, right)

        barrier = pltpu.get_barrier_semaphore()
        for nbr in (left, right):
            pl.semaphore_signal(barrier, inc=1, device_id=(nbr,),
                                device_id_type=pl.DeviceIdType.MESH)
        pl.semaphore_wait(barrier, 2)

        def cols(d):
            return pl.ds(d * half, half)

        seeds = []
        for d in (0, 1):
            ld = pltpu.make_async_copy(
                p_ref.at[pl.ds(rr * chunk, chunk), cols(d)],
                acc[d], load_sem[d])
            ld.start()
            seeds.append(ld)
        for ld in seeds:
            ld.wait()

        def make_rdma(d):
            return pltpu.make_async_remote_copy(
                src_ref=acc[d], dst_ref=recv[d],
                send_sem=send_sem[d], recv_sem=recv_sem[d],
                device_id=(nbr_send[d],),
                device_id_type=pl.DeviceIdType.MESH)

        def rs_step(s, carry):
            lds = []
            for d in (0, 1):
                idx = lax.rem(rr - SGN[d] * (s + 1) + 2 * N_DEV, N_DEV)
                ld = pltpu.make_async_copy(
                    p_ref.at[pl.ds(idx * chunk, chunk), cols(d)],
                    local[d], load_sem[d])
                ld.start()
                lds.append(ld)

            @pl.when(s > 0)
            def _():
                for d in (0, 1):
                    pl.semaphore_wait(credit[d], 1)

            rdmas = [make_rdma(d) for d in (0, 1)]
            for r in rdmas:
                r.start()
            for r in rdmas:
                r.wait()
            for d in (0, 1):
                lds[d].wait()
                acc[d][...] = recv[d][...] + local[d][...]
                pl.semaphore_signal(credit[d], inc=1,
                                    device_id=(nbr_recv[d],),
                                    device_id_type=pl.DeviceIdType.MESH)
            return carry

        lax.fori_loop(0, N_DEV - 1, rs_step, 0)

        sts = []
        for d in (0, 1):
            owned = lax.rem(rr + SGN[d] + N_DEV, N_DEV)
            st = pltpu.make_async_copy(
                acc[d], out_ref.at[pl.ds(owned * chunk, chunk), cols(d)],
                store_sem[d])
            st.start()
            sts.append(st)
        for st in sts:
            st.wait()

        def ag_step(s, carry):
            for d in (0, 1):
                pl.semaphore_wait(credit[d], 1)
            rdmas = [make_rdma(d) for d in (0, 1)]
            for r in rdmas:
                r.start()
            for r in rdmas:
                r.wait()
            sts = []
            for d in (0, 1):
                idx = lax.rem(rr - SGN[d] * s + 2 * N_DEV, N_DEV)
                st = pltpu.make_async_copy(
                    recv[d], out_ref.at[pl.ds(idx * chunk, chunk), cols(d)],
                    store_sem[d])
                st.start()
                acc[d][...] = recv[d][...]
                sts.append(st)
            for st in sts:
                st.wait()

            @pl.when(s < N_DEV - 2)
            def _():
                for d in (0, 1):
                    pl.semaphore_signal(credit[d], inc=1,
                                        device_id=(nbr_recv[d],),
                                        device_id_type=pl.DeviceIdType.MESH)
            return carry

        lax.fori_loop(0, N_DEV - 1, ag_step, 0)

    buf = lambda: pltpu.VMEM((chunk, half), jnp.float32)
    return pl.pallas_call(
        body,
        out_shape=jax.ShapeDtypeStruct((m, n), jnp.float32),
        in_specs=[
            pl.BlockSpec(memory_space=pltpu.SMEM),
            pl.BlockSpec(memory_space=pl.ANY),
        ],
        out_specs=pl.BlockSpec(memory_space=pl.ANY),
        scratch_shapes=[
            buf(), buf(), buf(),
            buf(), buf(), buf(),
            pltpu.SemaphoreType.DMA, pltpu.SemaphoreType.DMA,
            pltpu.SemaphoreType.DMA, pltpu.SemaphoreType.DMA,
            pltpu.SemaphoreType.DMA, pltpu.SemaphoreType.DMA,
            pltpu.SemaphoreType.DMA, pltpu.SemaphoreType.DMA,
            pltpu.SemaphoreType.REGULAR, pltpu.SemaphoreType.REGULAR,
        ],
        compiler_params=pltpu.CompilerParams(collective_id=0),
    )(meta, partial)


def kernel(x, w_mat):
    partial = jnp.dot(x, w_mat, preferred_element_type=jnp.float32,
                      precision=lax.Precision.HIGHEST)
    my = lax.axis_index("i")
    perm = jnp.asarray(PERM)
    inv = jnp.asarray(INV)
    r = inv[my]
    meta = jnp.stack([
        r,
        perm[lax.rem(r + N_DEV - 1, N_DEV)],
        perm[lax.rem(r + 1, N_DEV)],
    ]).astype(jnp.int32)
    y = _ring_all_reduce(partial, meta)
    amax = jnp.max(jnp.abs(y))
    scale = amax / 127.0
    q = jnp.clip(jnp.round(y / scale), -127.0, 127.0)
    return (q * scale).astype(jnp.float32)


# device time: 1782217 ns/iter; 1.9912x vs baseline; 1.0533x over previous
import numpy as np

import jax
import jax.numpy as jnp
from jax import lax
from jax.experimental import pallas as pl
from jax.experimental.pallas import tpu as pltpu

N_DEV = 32
SGN = (1, -1)


def _mesh_coord_order():
    coords = [(x, y, z) for x in (0, 1) for y in range(4) for z in range(4)]
    order = []
    for z in sorted({c[2] for c in coords}):
        plane = sorted(c for c in coords if c[2] == z)
        for yi, y in enumerate(sorted({c[1] for c in plane})):
            row = sorted((c for c in plane if c[1] == y), reverse=bool(yi % 2))
            order.extend(row)
    return order


def _ham_cycle():
    path = []
    for zi in range(4):
        ys = range(4) if zi % 2 == 0 else range(3, -1, -1)
        for y in ys:
            path.append((0, y, zi))
    path += [(1, y, z) for (_, y, z) in reversed(path)]
    return path


_MESH_ORDER = _mesh_coord_order()
_HAM = _ham_cycle()
for _k in range(N_DEV):
    _a, _b = _HAM[_k], _HAM[(_k + 1) % N_DEV]
    assert sum(abs(_a[_i] - _b[_i]) for _i in range(3)) == 1, (_k, _a, _b)
PERM = np.array([_MESH_ORDER.index(c) for c in _HAM], dtype=np.int32)
assert sorted(PERM.tolist()) == list(range(N_DEV))
INV = np.argsort(PERM).astype(np.int32)


def _ring_all_reduce(partial, meta):
    m, n = partial.shape
    chunk = m // N_DEV
    half = n // 2
    quarter = n // 4

    def body(meta_ref, p_ref, out_ref,
             acc0, recv0, local0, acc1, recv1, local1,
             send_sems0, send_sems1, recv_sems0, recv_sems1,
             load_sem0, load_sem1, store_sems0, store_sems1,
             credit00, credit01, credit10, credit11):
        acc = (acc0, acc1)
        recv = (recv0, recv1)
        local = (local0, local1)
        send_sems = (send_sems0, send_sems1)
        recv_sems = (recv_sems0, recv_sems1)
        load_sem = (load_sem0, load_sem1)
        store_sems = (store_sems0, store_sems1)
        credit = ((credit00, credit01), (credit10, credit11))

        rr = meta_ref[0]
        left = meta_ref[1]
        right = meta_ref[2]
        nbr_send = (right, left)
        nbr_recv = (left, right)

        barrier = pltpu.get_barrier_semaphore()
        for nbr in (left, right):
            pl.semaphore_signal(barrier, inc=1, device_id=(nbr,),
                                device_id_type=pl.DeviceIdType.MESH)
        pl.semaphore_wait(barrier, 2)

        def gcols(d):
            return pl.ds(d * half, half)

        def sub(u):
            return pl.ds(u * quarter, quarter)

        seeds = []
        for d in (0, 1):
            ld = pltpu.make_async_copy(
                p_ref.at[pl.ds(rr * chunk, chunk), gcols(d)],
                acc[d], load_sem[d])
            ld.start()
            seeds.append(ld)
        for ld in seeds:
            ld.wait()

        def make_rdma(d, u):
            return pltpu.make_async_remote_copy(
                src_ref=acc[d].at[:, sub(u)],
                dst_ref=recv[d].at[:, sub(u)],
                send_sem=send_sems[d].at[u],
                recv_sem=recv_sems[d].at[u],
                device_id=(nbr_send[d],),
                device_id_type=pl.DeviceIdType.MESH)

        def rs_step(s, carry):
            lds = []
            for d in (0, 1):
                idx = lax.rem(rr - SGN[d] * (s + 1) + 2 * N_DEV, N_DEV)
                ld = pltpu.make_async_copy(
                    p_ref.at[pl.ds(idx * chunk, chunk), gcols(d)],
                    local[d], load_sem[d])
                ld.start()
                lds.append(ld)

            @pl.when(s > 0)
            def _():
                for d in (0, 1):
                    for u in (0, 1):
                        pl.semaphore_wait(credit[d][u], 1)

            rd = {}
            for u in (0, 1):
                for d in (0, 1):
                    rd[d, u] = make_rdma(d, u)
                    rd[d, u].start()
            for d in (0, 1):
                lds[d].wait()
            for u in (0, 1):
                for d in (0, 1):
                    rd[d, u].wait()
                for d in (0, 1):
                    acc[d][:, sub(u)] = recv[d][:, sub(u)] + local[d][:, sub(u)]
            for d in (0, 1):
                for u in (0, 1):
                    pl.semaphore_signal(credit[d][u], inc=1,
                                        device_id=(nbr_recv[d],),
                                        device_id_type=pl.DeviceIdType.MESH)
            return carry

        lax.fori_loop(0, N_DEV - 1, rs_step, 0)

        sts = []
        for d in (0, 1):
            owned = lax.rem(rr + SGN[d] + N_DEV, N_DEV)
            st = pltpu.make_async_copy(
                acc[d], out_ref.at[pl.ds(owned * chunk, chunk), gcols(d)],
                store_sems[d].at[0])
            st.start()
            sts.append(st)
        for st in sts:
            st.wait()

        def ag_step(s, carry):
            for d in (0, 1):
                for u in (0, 1):
                    pl.semaphore_wait(credit[d][u], 1)
            rd = {}
            for u in (0, 1):
                for d in (0, 1):
                    rd[d, u] = make_rdma(d, u)
                    rd[d, u].start()
            for u in (0, 1):
                for d in (0, 1):
                    rd[d, u].wait()
                sts = []
                for d in (0, 1):
                    idx = lax.rem(rr - SGN[d] * s + 2 * N_DEV, N_DEV)
                    st = pltpu.make_async_copy(
                        recv[d].at[:, sub(u)],
                        out_ref.at[pl.ds(idx * chunk, chunk),
                                   pl.ds(d * half + u * quarter, quarter)],
                        store_sems[d].at[u])
                    st.start()
                    sts.append(st)
                for d in (0, 1):
                    acc[d][:, sub(u)] = recv[d][:, sub(u)]
                for st in sts:
                    st.wait()
                @pl.when(s < N_DEV - 2)
                def _():
                    for d in (0, 1):
                        pl.semaphore_signal(credit[d][u], inc=1,
                                            device_id=(nbr_recv[d],),
                                            device_id_type=pl.DeviceIdType.MESH)
            return carry

        lax.fori_loop(0, N_DEV - 1, ag_step, 0)

    buf = lambda: pltpu.VMEM((chunk, half), jnp.float32)
    return pl.pallas_call(
        body,
        out_shape=jax.ShapeDtypeStruct((m, n), jnp.float32),
        in_specs=[
            pl.BlockSpec(memory_space=pltpu.SMEM),
            pl.BlockSpec(memory_space=pl.ANY),
        ],
        out_specs=pl.BlockSpec(memory_space=pl.ANY),
        scratch_shapes=[
            buf(), buf(), buf(),
            buf(), buf(), buf(),
            pltpu.SemaphoreType.DMA((2,)), pltpu.SemaphoreType.DMA((2,)),
            pltpu.SemaphoreType.DMA((2,)), pltpu.SemaphoreType.DMA((2,)),
            pltpu.SemaphoreType.DMA, pltpu.SemaphoreType.DMA,
            pltpu.SemaphoreType.DMA((2,)), pltpu.SemaphoreType.DMA((2,)),
            pltpu.SemaphoreType.REGULAR, pltpu.SemaphoreType.REGULAR,
            pltpu.SemaphoreType.REGULAR, pltpu.SemaphoreType.REGULAR,
        ],
        compiler_params=pltpu.CompilerParams(collective_id=0),
    )(meta, partial)


def kernel(x, w_mat):
    partial = jnp.dot(x, w_mat, preferred_element_type=jnp.float32)
    my = lax.axis_index("i")
    perm = jnp.asarray(PERM)
    inv = jnp.asarray(INV)
    r = inv[my]
    meta = jnp.stack([
        r,
        perm[lax.rem(r + N_DEV - 1, N_DEV)],
        perm[lax.rem(r + 1, N_DEV)],
    ]).astype(jnp.int32)
    y = _ring_all_reduce(partial, meta)
    amax = jnp.max(jnp.abs(y))
    scale = amax / 127.0
    q = jnp.clip(jnp.round(y / scale), -127.0, 127.0)
    return (q * scale).astype(jnp.float32)


# device time: 1586110 ns/iter; 2.2374x vs baseline; 1.1236x over previous
import numpy as np

import jax
import jax.numpy as jnp
from jax import lax
from jax.experimental import pallas as pl
from jax.experimental.pallas import tpu as pltpu

N_DEV = 32
SGN = (1, -1)


def _mesh_coord_order():
    coords = [(x, y, z) for x in (0, 1) for y in range(4) for z in range(4)]
    order = []
    for z in sorted({c[2] for c in coords}):
        plane = sorted(c for c in coords if c[2] == z)
        for yi, y in enumerate(sorted({c[1] for c in plane})):
            row = sorted((c for c in plane if c[1] == y), reverse=bool(yi % 2))
            order.extend(row)
    return order


def _ham_cycle():
    path = []
    for zi in range(4):
        ys = range(4) if zi % 2 == 0 else range(3, -1, -1)
        for y in ys:
            path.append((0, y, zi))
    path += [(1, y, z) for (_, y, z) in reversed(path)]
    return path


_MESH_ORDER = _mesh_coord_order()
_HAM = _ham_cycle()
for _k in range(N_DEV):
    _a, _b = _HAM[_k], _HAM[(_k + 1) % N_DEV]
    assert sum(abs(_a[_i] - _b[_i]) for _i in range(3)) == 1, (_k, _a, _b)
PERM = np.array([_MESH_ORDER.index(c) for c in _HAM], dtype=np.int32)
assert sorted(PERM.tolist()) == list(range(N_DEV))
INV = np.argsort(PERM).astype(np.int32)


def _ring_all_reduce(partial, meta):
    m, n = partial.shape
    chunk = m // N_DEV
    half = n // 2
    quarter = n // 4

    def body(meta_ref, p_ref, out_ref,
             acc0, recv0, local0, acc1, recv1, local1,
             send_sems0, send_sems1, recv_sems0, recv_sems1,
             load_sem0, load_sem1, store_sems0, store_sems1,
             credit00, credit01, credit10, credit11):
        acc = (acc0, acc1)
        recv = (recv0, recv1)
        local = (local0, local1)
        send_sems = (send_sems0, send_sems1)
        recv_sems = (recv_sems0, recv_sems1)
        load_sem = (load_sem0, load_sem1)
        store_sems = (store_sems0, store_sems1)
        credit = ((credit00, credit01), (credit10, credit11))

        rr = meta_ref[0]
        left = meta_ref[1]
        right = meta_ref[2]
        nbr_send = (right, left)
        nbr_recv = (left, right)

        barrier = pltpu.get_barrier_semaphore()
        for nbr in (left, right):
            pl.semaphore_signal(barrier, inc=1, device_id=(nbr,),
                                device_id_type=pl.DeviceIdType.MESH)
        pl.semaphore_wait(barrier, 2)

        def gcols(d):
            return pl.ds(d * half, half)

        def sub(u):
            return pl.ds(u * quarter, quarter)

        def make_rdma(d, u):
            return pltpu.make_async_remote_copy(
                src_ref=acc[d].at[:, sub(u)],
                dst_ref=recv[d].at[:, sub(u)],
                send_sem=send_sems[d].at[u],
                recv_sem=recv_sems[d].at[u],
                device_id=(nbr_send[d],),
                device_id_type=pl.DeviceIdType.MESH)

        seeds = []
        for d in (0, 1):
            ld = pltpu.make_async_copy(
                p_ref.at[pl.ds(rr * chunk, chunk), gcols(d)],
                acc[d], load_sem[d])
            ld.start()
            seeds.append(ld)
        for ld in seeds:
            ld.wait()

        for u in (0, 1):
            for d in (0, 1):
                make_rdma(d, u).start()

        def rs_step(s, carry):
            lds = []
            for d in (0, 1):
                idx = lax.rem(rr - SGN[d] * (s + 1) + 2 * N_DEV, N_DEV)
                ld = pltpu.make_async_copy(
                    p_ref.at[pl.ds(idx * chunk, chunk), gcols(d)],
                    local[d], load_sem[d])
                ld.start()
                lds.append(ld)

            for u in (0, 1):
                rdw = []
                for d in (0, 1):
                    r_ = make_rdma(d, u)
                    r_.wait()
                    rdw.append(r_)
                if u == 0:
                    for d in (0, 1):
                        lds[d].wait()
                for d in (0, 1):
                    acc[d][:, sub(u)] = recv[d][:, sub(u)] + local[d][:, sub(u)]
                for d in (0, 1):
                    pl.semaphore_signal(credit[d][u], inc=1,
                                        device_id=(nbr_recv[d],),
                                        device_id_type=pl.DeviceIdType.MESH)

                @pl.when(s < N_DEV - 2)
                def _():
                    for d in (0, 1):
                        pl.semaphore_wait(credit[d][u], 1)
                        make_rdma(d, u).start()
            return carry

        lax.fori_loop(0, N_DEV - 1, rs_step, 0)

        for u in (0, 1):
            for d in (0, 1):
                pl.semaphore_wait(credit[d][u], 1)
                make_rdma(d, u).start()
        sts = []
        for d in (0, 1):
            owned = lax.rem(rr + SGN[d] + N_DEV, N_DEV)
            st = pltpu.make_async_copy(
                acc[d], out_ref.at[pl.ds(owned * chunk, chunk), gcols(d)],
                store_sems[d].at[0])
            st.start()
            sts.append(st)
        for st in sts:
            st.wait()

        def ag_step(s, carry):
            for u in (0, 1):
                for d in (0, 1):
                    make_rdma(d, u).wait()
                sts = []
                for d in (0, 1):
                    idx = lax.rem(rr - SGN[d] * s + 2 * N_DEV, N_DEV)
                    st = pltpu.make_async_copy(
                        recv[d].at[:, sub(u)],
                        out_ref.at[pl.ds(idx * chunk, chunk),
                                   pl.ds(d * half + u * quarter, quarter)],
                        store_sems[d].at[u])
                    st.start()
                    sts.append(st)
                for d in (0, 1):
                    acc[d][:, sub(u)] = recv[d][:, sub(u)]
                for st in sts:
                    st.wait()

                @pl.when(s < N_DEV - 2)
                def _():
                    for d in (0, 1):
                        pl.semaphore_signal(credit[d][u], inc=1,
                                            device_id=(nbr_recv[d],),
                                            device_id_type=pl.DeviceIdType.MESH)
                    for d in (0, 1):
                        pl.semaphore_wait(credit[d][u], 1)
                        make_rdma(d, u).start()
            return carry

        lax.fori_loop(0, N_DEV - 1, ag_step, 0)

    buf = lambda: pltpu.VMEM((chunk, half), jnp.float32)
    return pl.pallas_call(
        body,
        out_shape=jax.ShapeDtypeStruct((m, n), jnp.float32),
        in_specs=[
            pl.BlockSpec(memory_space=pltpu.SMEM),
            pl.BlockSpec(memory_space=pl.ANY),
        ],
        out_specs=pl.BlockSpec(memory_space=pl.ANY),
        scratch_shapes=[
            buf(), buf(), buf(),
            buf(), buf(), buf(),
            pltpu.SemaphoreType.DMA((2,)), pltpu.SemaphoreType.DMA((2,)),
            pltpu.SemaphoreType.DMA((2,)), pltpu.SemaphoreType.DMA((2,)),
            pltpu.SemaphoreType.DMA, pltpu.SemaphoreType.DMA,
            pltpu.SemaphoreType.DMA((2,)), pltpu.SemaphoreType.DMA((2,)),
            pltpu.SemaphoreType.REGULAR, pltpu.SemaphoreType.REGULAR,
            pltpu.SemaphoreType.REGULAR, pltpu.SemaphoreType.REGULAR,
        ],
        compiler_params=pltpu.CompilerParams(collective_id=0),
    )(meta, partial)


def kernel(x, w_mat):
    partial = jnp.dot(x, w_mat, preferred_element_type=jnp.float32)
    my = lax.axis_index("i")
    perm = jnp.asarray(PERM)
    inv = jnp.asarray(INV)
    r = inv[my]
    meta = jnp.stack([
        r,
        perm[lax.rem(r + N_DEV - 1, N_DEV)],
        perm[lax.rem(r + 1, N_DEV)],
    ]).astype(jnp.int32)
    y = _ring_all_reduce(partial, meta)
    amax = jnp.max(jnp.abs(y))
    scale = amax / 127.0
    q = jnp.clip(jnp.round(y / scale), -127.0, 127.0)
    return (q * scale).astype(jnp.float32)


# device time: 1544519 ns/iter; 2.2976x vs baseline; 1.0269x over previous
import numpy as np

import jax
import jax.numpy as jnp
from jax import lax
from jax.experimental import pallas as pl
from jax.experimental.pallas import tpu as pltpu

N_DEV = 32
SGN = (1, -1)


def _mesh_coord_order():
    coords = [(x, y, z) for x in (0, 1) for y in range(4) for z in range(4)]
    order = []
    for z in sorted({c[2] for c in coords}):
        plane = sorted(c for c in coords if c[2] == z)
        for yi, y in enumerate(sorted({c[1] for c in plane})):
            row = sorted((c for c in plane if c[1] == y), reverse=bool(yi % 2))
            order.extend(row)
    return order


def _ham_cycle():
    path = []
    for zi in range(4):
        ys = range(4) if zi % 2 == 0 else range(3, -1, -1)
        for y in ys:
            path.append((0, y, zi))
    path += [(1, y, z) for (_, y, z) in reversed(path)]
    return path


_MESH_ORDER = _mesh_coord_order()
_HAM = _ham_cycle()
for _k in range(N_DEV):
    _a, _b = _HAM[_k], _HAM[(_k + 1) % N_DEV]
    assert sum(abs(_a[_i] - _b[_i]) for _i in range(3)) == 1, (_k, _a, _b)
PERM = np.array([_MESH_ORDER.index(c) for c in _HAM], dtype=np.int32)
assert sorted(PERM.tolist()) == list(range(N_DEV))
INV = np.argsort(PERM).astype(np.int32)


def _ring_all_reduce(partial, meta):
    m, n = partial.shape
    chunk = m // N_DEV
    half = n // 2
    quarter = n // 4

    def body(meta_ref, p_ref, out_ref, amax_ref,
             acc0, recv0, local0, acc1, recv1, local1,
             send_sems0, send_sems1, recv_sems0, recv_sems1,
             load_sem0, load_sem1, store_sems0, store_sems1,
             credit00, credit01, credit10, credit11):
        acc = (acc0, acc1)
        recv = (recv0, recv1)
        local = (local0, local1)
        send_sems = (send_sems0, send_sems1)
        recv_sems = (recv_sems0, recv_sems1)
        load_sem = (load_sem0, load_sem1)
        store_sems = (store_sems0, store_sems1)
        credit = ((credit00, credit01), (credit10, credit11))

        rr = meta_ref[0]
        left = meta_ref[1]
        right = meta_ref[2]
        nbr_send = (right, left)
        nbr_recv = (left, right)

        barrier = pltpu.get_barrier_semaphore()
        for nbr in (left, right):
            pl.semaphore_signal(barrier, inc=1, device_id=(nbr,),
                                device_id_type=pl.DeviceIdType.MESH)
        pl.semaphore_wait(barrier, 2)

        def gcols(d):
            return pl.ds(d * half, half)

        def sub(u):
            return pl.ds(u * quarter, quarter)

        def make_rdma(d, u):
            return pltpu.make_async_remote_copy(
                src_ref=acc[d].at[:, sub(u)],
                dst_ref=recv[d].at[:, sub(u)],
                send_sem=send_sems[d].at[u],
                recv_sem=recv_sems[d].at[u],
                device_id=(nbr_send[d],),
                device_id_type=pl.DeviceIdType.MESH)

        seeds = []
        for d in (0, 1):
            ld = pltpu.make_async_copy(
                p_ref.at[pl.ds(rr * chunk, chunk), gcols(d)],
                acc[d], load_sem[d])
            ld.start()
            seeds.append(ld)
        for ld in seeds:
            ld.wait()

        for u in (0, 1):
            for d in (0, 1):
                make_rdma(d, u).start()

        def rs_step(s, carry):
            lds = []
            for d in (0, 1):
                idx = lax.rem(rr - SGN[d] * (s + 1) + 2 * N_DEV, N_DEV)
                ld = pltpu.make_async_copy(
                    p_ref.at[pl.ds(idx * chunk, chunk), gcols(d)],
                    local[d], load_sem[d])
                ld.start()
                lds.append(ld)

            for u in (0, 1):
                rdw = []
                for d in (0, 1):
                    r_ = make_rdma(d, u)
                    r_.wait()
                    rdw.append(r_)
                if u == 0:
                    for d in (0, 1):
                        lds[d].wait()
                for d in (0, 1):
                    acc[d][:, sub(u)] = recv[d][:, sub(u)] + local[d][:, sub(u)]
                for d in (0, 1):
                    pl.semaphore_signal(credit[d][u], inc=1,
                                        device_id=(nbr_recv[d],),
                                        device_id_type=pl.DeviceIdType.MESH)

                @pl.when(s < N_DEV - 2)
                def _():
                    for d in (0, 1):
                        pl.semaphore_wait(credit[d][u], 1)
                        make_rdma(d, u).start()
            return carry

        lax.fori_loop(0, N_DEV - 1, rs_step, 0)

        for u in (0, 1):
            for d in (0, 1):
                pl.semaphore_wait(credit[d][u], 1)
                make_rdma(d, u).start()
        amax_ref[0] = jnp.maximum(jnp.max(jnp.abs(acc[0][...])),
                                  jnp.max(jnp.abs(acc[1][...])))
        sts = []
        for d in (0, 1):
            owned = lax.rem(rr + SGN[d] + N_DEV, N_DEV)
            st = pltpu.make_async_copy(
                acc[d], out_ref.at[pl.ds(owned * chunk, chunk), gcols(d)],
                store_sems[d].at[0])
            st.start()
            sts.append(st)
        for st in sts:
            st.wait()

        def ag_step(s, carry):
            for u in (0, 1):
                for d in (0, 1):
                    make_rdma(d, u).wait()
                sts = []
                for d in (0, 1):
                    idx = lax.rem(rr - SGN[d] * s + 2 * N_DEV, N_DEV)
                    st = pltpu.make_async_copy(
                        recv[d].at[:, sub(u)],
                        out_ref.at[pl.ds(idx * chunk, chunk),
                                   pl.ds(d * half + u * quarter, quarter)],
                        store_sems[d].at[u])
                    st.start()
                    sts.append(st)
                for d in (0, 1):
                    acc[d][:, sub(u)] = recv[d][:, sub(u)]
                    amax_ref[0] = jnp.maximum(
                        amax_ref[0], jnp.max(jnp.abs(recv[d][:, sub(u)])))
                for st in sts:
                    st.wait()

                @pl.when(s < N_DEV - 2)
                def _():
                    for d in (0, 1):
                        pl.semaphore_signal(credit[d][u], inc=1,
                                            device_id=(nbr_recv[d],),
                                            device_id_type=pl.DeviceIdType.MESH)
                    for d in (0, 1):
                        pl.semaphore_wait(credit[d][u], 1)
                        make_rdma(d, u).start()
            return carry

        lax.fori_loop(0, N_DEV - 1, ag_step, 0)

    buf = lambda: pltpu.VMEM((chunk, half), jnp.float32)
    return pl.pallas_call(
        body,
        out_shape=[
            jax.ShapeDtypeStruct((m, n), jnp.float32),
            jax.ShapeDtypeStruct((1,), jnp.float32),
        ],
        in_specs=[
            pl.BlockSpec(memory_space=pltpu.SMEM),
            pl.BlockSpec(memory_space=pl.ANY),
        ],
        out_specs=[
            pl.BlockSpec(memory_space=pl.ANY),
            pl.BlockSpec(memory_space=pltpu.SMEM),
        ],
        scratch_shapes=[
            buf(), buf(), buf(),
            buf(), buf(), buf(),
            pltpu.SemaphoreType.DMA((2,)), pltpu.SemaphoreType.DMA((2,)),
            pltpu.SemaphoreType.DMA((2,)), pltpu.SemaphoreType.DMA((2,)),
            pltpu.SemaphoreType.DMA, pltpu.SemaphoreType.DMA,
            pltpu.SemaphoreType.DMA((2,)), pltpu.SemaphoreType.DMA((2,)),
            pltpu.SemaphoreType.REGULAR, pltpu.SemaphoreType.REGULAR,
            pltpu.SemaphoreType.REGULAR, pltpu.SemaphoreType.REGULAR,
        ],
        compiler_params=pltpu.CompilerParams(collective_id=0),
    )(meta, partial)


def kernel(x, w_mat):
    partial = jnp.dot(x, w_mat, preferred_element_type=jnp.float32)
    my = lax.axis_index("i")
    perm = jnp.asarray(PERM)
    inv = jnp.asarray(INV)
    r = inv[my]
    meta = jnp.stack([
        r,
        perm[lax.rem(r + N_DEV - 1, N_DEV)],
        perm[lax.rem(r + 1, N_DEV)],
    ]).astype(jnp.int32)
    y, amax_arr = _ring_all_reduce(partial, meta)
    scale = amax_arr[0] / 127.0
    q = jnp.clip(jnp.round(y / scale), -127.0, 127.0)
    return (q * scale).astype(jnp.float32)


# device time: 1501476 ns/iter; 2.3635x vs baseline; 1.0287x over previous
import numpy as np

import jax
import jax.numpy as jnp
from jax import lax
from jax.experimental import pallas as pl
from jax.experimental.pallas import tpu as pltpu

N_DEV = 32
SGN = (1, -1)


def _mesh_coord_order():
    coords = [(x, y, z) for x in (0, 1) for y in range(4) for z in range(4)]
    order = []
    for z in sorted({c[2] for c in coords}):
        plane = sorted(c for c in coords if c[2] == z)
        for yi, y in enumerate(sorted({c[1] for c in plane})):
            row = sorted((c for c in plane if c[1] == y), reverse=bool(yi % 2))
            order.extend(row)
    return order


def _ham_cycle():
    path = []
    for zi in range(4):
        ys = range(4) if zi % 2 == 0 else range(3, -1, -1)
        for y in ys:
            path.append((0, y, zi))
    path += [(1, y, z) for (_, y, z) in reversed(path)]
    return path


_MESH_ORDER = _mesh_coord_order()
_HAM = _ham_cycle()
for _k in range(N_DEV):
    _a, _b = _HAM[_k], _HAM[(_k + 1) % N_DEV]
    assert sum(abs(_a[_i] - _b[_i]) for _i in range(3)) == 1, (_k, _a, _b)
PERM = np.array([_MESH_ORDER.index(c) for c in _HAM], dtype=np.int32)
assert sorted(PERM.tolist()) == list(range(N_DEV))
INV = np.argsort(PERM).astype(np.int32)


def _gemm_all_reduce(x, w_mat, meta):
    m = x.shape[0]
    n = w_mat.shape[1]
    chunk = m // N_DEV
    half = n // 2
    quarter = n // 4

    def body(meta_ref, x_ref, w_ref, out_ref, amax_ref,
             acc0, recv0, local0, acc1, recv1, local1,
             send_sems0, send_sems1, recv_sems0, recv_sems1,
             store_sems0, store_sems1,
             credit00, credit01, credit10, credit11):
        acc = (acc0, acc1)
        recv = (recv0, recv1)
        local = (local0, local1)
        send_sems = (send_sems0, send_sems1)
        recv_sems = (recv_sems0, recv_sems1)
        store_sems = (store_sems0, store_sems1)
        credit = ((credit00, credit01), (credit10, credit11))

        rr = meta_ref[0]
        left = meta_ref[1]
        right = meta_ref[2]
        nbr_send = (right, left)
        nbr_recv = (left, right)

        barrier = pltpu.get_barrier_semaphore()
        for nbr in (left, right):
            pl.semaphore_signal(barrier, inc=1, device_id=(nbr,),
                                device_id_type=pl.DeviceIdType.MESH)
        pl.semaphore_wait(barrier, 2)

        def gcols(d):
            return pl.ds(d * half, half)

        def sub(u):
            return pl.ds(u * quarter, quarter)

        def make_rdma(d, u):
            return pltpu.make_async_remote_copy(
                src_ref=acc[d].at[:, sub(u)],
                dst_ref=recv[d].at[:, sub(u)],
                send_sem=send_sems[d].at[u],
                recv_sem=recv_sems[d].at[u],
                device_id=(nbr_send[d],),
                device_id_type=pl.DeviceIdType.MESH)

        def partial_chunk(idx, d):
            return jnp.dot(x_ref[pl.ds(idx * chunk, chunk), :],
                           w_ref[:, gcols(d)],
                           preferred_element_type=jnp.float32)

        for d in (0, 1):
            acc[d][...] = partial_chunk(rr, d)

        for u in (0, 1):
            for d in (0, 1):
                make_rdma(d, u).start()

        def rs_step(s, carry):
            for d in (0, 1):
                idx = lax.rem(rr - SGN[d] * (s + 1) + 2 * N_DEV, N_DEV)
                local[d][...] = partial_chunk(idx, d)

            for u in (0, 1):
                for d in (0, 1):
                    make_rdma(d, u).wait()
                for d in (0, 1):
                    acc[d][:, sub(u)] = recv[d][:, sub(u)] + local[d][:, sub(u)]
                for d in (0, 1):
                    pl.semaphore_signal(credit[d][u], inc=1,
                                        device_id=(nbr_recv[d],),
                                        device_id_type=pl.DeviceIdType.MESH)

                @pl.when(s < N_DEV - 2)
                def _():
                    for d in (0, 1):
                        pl.semaphore_wait(credit[d][u], 1)
                        make_rdma(d, u).start()
            return carry

        lax.fori_loop(0, N_DEV - 1, rs_step, 0)

        for u in (0, 1):
            for d in (0, 1):
                pl.semaphore_wait(credit[d][u], 1)
                make_rdma(d, u).start()
        amax_ref[0] = jnp.maximum(jnp.max(jnp.abs(acc[0][...])),
                                  jnp.max(jnp.abs(acc[1][...])))
        sts = []
        for d in (0, 1):
            owned = lax.rem(rr + SGN[d] + N_DEV, N_DEV)
            st = pltpu.make_async_copy(
                acc[d], out_ref.at[pl.ds(owned * chunk, chunk), gcols(d)],
                store_sems[d].at[0])
            st.start()
            sts.append(st)
        for st in sts:
            st.wait()

        def ag_step(s, carry):
            for u in (0, 1):
                for d in (0, 1):
                    make_rdma(d, u).wait()
                sts = []
                for d in (0, 1):
                    idx = lax.rem(rr - SGN[d] * s + 2 * N_DEV, N_DEV)
                    st = pltpu.make_async_copy(
                        recv[d].at[:, sub(u)],
                        out_ref.at[pl.ds(idx * chunk, chunk),
                                   pl.ds(d * half + u * quarter, quarter)],
                        store_sems[d].at[u])
                    st.start()
                    sts.append(st)
                for d in (0, 1):
                    acc[d][:, sub(u)] = recv[d][:, sub(u)]
                    amax_ref[0] = jnp.maximum(
                        amax_ref[0], jnp.max(jnp.abs(recv[d][:, sub(u)])))
                for st in sts:
                    st.wait()

                @pl.when(s < N_DEV - 2)
                def _():
                    for d in (0, 1):
                        pl.semaphore_signal(credit[d][u], inc=1,
                                            device_id=(nbr_recv[d],),
                                            device_id_type=pl.DeviceIdType.MESH)
                    for d in (0, 1):
                        pl.semaphore_wait(credit[d][u], 1)
                        make_rdma(d, u).start()
            return carry

        lax.fori_loop(0, N_DEV - 1, ag_step, 0)

    buf = lambda: pltpu.VMEM((chunk, half), jnp.float32)
    return pl.pallas_call(
        body,
        out_shape=[
            jax.ShapeDtypeStruct((m, n), jnp.float32),
            jax.ShapeDtypeStruct((1,), jnp.float32),
        ],
        in_specs=[
            pl.BlockSpec(memory_space=pltpu.SMEM),
            pl.BlockSpec(memory_space=pltpu.VMEM),
            pl.BlockSpec(memory_space=pltpu.VMEM),
        ],
        out_specs=[
            pl.BlockSpec(memory_space=pl.ANY),
            pl.BlockSpec(memory_space=pltpu.SMEM),
        ],
        scratch_shapes=[
            buf(), buf(), buf(),
            buf(), buf(), buf(),
            pltpu.SemaphoreType.DMA((2,)), pltpu.SemaphoreType.DMA((2,)),
            pltpu.SemaphoreType.DMA((2,)), pltpu.SemaphoreType.DMA((2,)),
            pltpu.SemaphoreType.DMA((2,)), pltpu.SemaphoreType.DMA((2,)),
            pltpu.SemaphoreType.REGULAR, pltpu.SemaphoreType.REGULAR,
            pltpu.SemaphoreType.REGULAR, pltpu.SemaphoreType.REGULAR,
        ],
        compiler_params=pltpu.CompilerParams(collective_id=0),
    )(meta, x, w_mat)


def kernel(x, w_mat):
    my = lax.axis_index("i")
    perm = jnp.asarray(PERM)
    inv = jnp.asarray(INV)
    r = inv[my]
    meta = jnp.stack([
        r,
        perm[lax.rem(r + N_DEV - 1, N_DEV)],
        perm[lax.rem(r + 1, N_DEV)],
    ]).astype(jnp.int32)
    y, amax_arr = _gemm_all_reduce(x, w_mat, meta)
    scale = amax_arr[0] / 127.0
    q = jnp.clip(jnp.round(y / scale), -127.0, 127.0)
    return (q * scale).astype(jnp.float32)
